# baseline (speedup 1.0000x reference)
import jax
import jax.numpy as jnp
from jax.experimental import pallas as pl
from jax.experimental.pallas import tpu as pltpu

_NV = 8
_EPS = 1e-5


def _compute(xb, ln_g, ln_b, W1, b1, W2, b2, gW, gb):
    mu = jnp.mean(xb, axis=1)
    msq = jnp.mean(xb * xb, axis=1)
    r = jax.lax.rsqrt(msq - mu * mu + _EPS)
    wsum = jnp.sum(xb * r[:, None, :], axis=0)
    corr = jnp.mean(mu * r, axis=0, keepdims=True)
    s = ln_g * (wsum * (1.0 / _NV) - corr) + ln_b

    h1 = jax.lax.dot_general(
        W1, s, (((0,), (0,)), ((), ())),
        preferred_element_type=jnp.float32) + b1

    mu2 = jnp.mean(h1, axis=0, keepdims=True)
    var2 = jnp.mean((h1 - mu2) ** 2, axis=0, keepdims=True)
    a = jnp.maximum((h1 - mu2) * jax.lax.rsqrt(var2 + _EPS), 0.0)

    h2 = jax.lax.dot_general(
        W2, a, (((0,), (0,)), ((), ())),
        preferred_element_type=jnp.float32) + b2

    pooled = jnp.mean(xb, axis=2)
    logits = jax.lax.dot_general(
        pooled, gW, (((1,), (1,)), ((), ())),
        preferred_element_type=jnp.float32) + gb
    gate = jax.nn.sigmoid(logits)

    return xb + gate[:, :, None] * h2[None, :, :]


def _body(x_hbm, ln_g_ref, ln_b_ref, W1_ref, b1_ref, W2_ref, b2_ref,
          gW_ref, gb_ref, out_hbm, xbuf, obuf, in_sem, out_sem):
    i = pl.program_id(0)
    n = pl.num_programs(0)
    slot = jax.lax.rem(i, 2)
    nslot = jax.lax.rem(i + 1, 2)

    def in_copy(blk, sl):
        return pltpu.make_async_copy(
            x_hbm.at[pl.ds(blk * _NV, _NV)], xbuf.at[sl], in_sem.at[sl])

    def out_copy(blk, sl):
        return pltpu.make_async_copy(
            obuf.at[sl], out_hbm.at[pl.ds(blk * _NV, _NV)], out_sem.at[sl])

    @pl.when(i == 0)
    def _():
        in_copy(0, 0).start()

    @pl.when(i + 1 < n)
    def _():
        in_copy(i + 1, nslot).start()

    in_copy(i, slot).wait()

    res = _compute(xbuf[slot], ln_g_ref[...], ln_b_ref[...], W1_ref[...],
                   b1_ref[...], W2_ref[...], b2_ref[...], gW_ref[...],
                   gb_ref[...])

    @pl.when(i >= 2)
    def _():
        out_copy(i - 2, slot).wait()

    obuf[slot] = res
    out_copy(i, slot).start()

    @pl.when(i == n - 1)
    def _():
        out_copy(i - 1, nslot).wait()
        out_copy(i, slot).wait()


@jax.jit
def kernel(x, data_key, ln_g, ln_b, W1, b1, W2, b2, gate_W, gate_b):
    B, C, T = x.shape
    n_groups = B // _NV

    in_specs = [
        pl.BlockSpec(memory_space=pl.ANY),
        pl.BlockSpec((C, 1), lambda i: (0, 0)),
        pl.BlockSpec((C, 1), lambda i: (0, 0)),
        pl.BlockSpec((C, C), lambda i: (0, 0)),
        pl.BlockSpec((C, 1), lambda i: (0, 0)),
        pl.BlockSpec((C, C), lambda i: (0, 0)),
        pl.BlockSpec((C, 1), lambda i: (0, 0)),
        pl.BlockSpec((C, C), lambda i: (0, 0)),
        pl.BlockSpec((1, C), lambda i: (0, 0)),
    ]

    return pl.pallas_call(
        _body,
        grid=(n_groups,),
        in_specs=in_specs,
        out_specs=pl.BlockSpec(memory_space=pl.ANY),
        out_shape=jax.ShapeDtypeStruct((B, C, T), x.dtype),
        scratch_shapes=[
            pltpu.VMEM((2, _NV, C, T), jnp.float32),
            pltpu.VMEM((2, _NV, C, T), jnp.float32),
            pltpu.SemaphoreType.DMA((2,)),
            pltpu.SemaphoreType.DMA((2,)),
        ],
        compiler_params=pltpu.CompilerParams(
            dimension_semantics=("arbitrary",),
        ),
    )(x, ln_g.reshape(C, 1), ln_b.reshape(C, 1), W1, b1.reshape(C, 1),
      W2, b2.reshape(C, 1), gate_W, gate_b.reshape(1, C))


# speedup vs baseline: 1.0444x; 1.0444x over previous
import jax
import jax.numpy as jnp
from jax.experimental import pallas as pl
from jax.experimental.pallas import tpu as pltpu

_NV = 8
_EPS = 1e-5


def _fused_body(x_ref, ln_g_ref, ln_b_ref, W1_ref, b1_ref, W2_ref, b2_ref,
                gW_ref, gb_ref, out_ref):
    ln_g = ln_g_ref[...]
    ln_b = ln_b_ref[...]
    gb = gb_ref[...]

    wsum = None
    corr = None
    gates = []
    for v in range(_NV):
        xv = x_ref[v]
        mu_v = jnp.mean(xv, axis=0, keepdims=True)
        msq_v = jnp.mean(xv * xv, axis=0, keepdims=True)
        r_v = jax.lax.rsqrt(msq_v - mu_v * mu_v + _EPS)
        term = xv * r_v
        cterm = mu_v * r_v
        wsum = term if wsum is None else wsum + term
        corr = cterm if corr is None else corr + cterm
        pooled_v = jnp.mean(xv, axis=1, keepdims=True)
        logit_v = jax.lax.dot_general(
            gW_ref[...], pooled_v, (((1,), (0,)), ((), ())),
            preferred_element_type=jnp.float32) + gb
        gates.append(jax.nn.sigmoid(logit_v))

    s = ln_g * ((wsum - corr) * (1.0 / _NV)) + ln_b

    h1 = jax.lax.dot_general(
        W1_ref[...], s, (((0,), (0,)), ((), ())),
        preferred_element_type=jnp.float32) + b1_ref[...]

    mu2 = jnp.mean(h1, axis=0, keepdims=True)
    var2 = jnp.mean((h1 - mu2) ** 2, axis=0, keepdims=True)
    a = jnp.maximum((h1 - mu2) * jax.lax.rsqrt(var2 + _EPS), 0.0)

    h2 = jax.lax.dot_general(
        W2_ref[...], a, (((0,), (0,)), ((), ())),
        preferred_element_type=jnp.float32) + b2_ref[...]

    for v in range(_NV):
        out_ref[v] = x_ref[v] + gates[v] * h2


@jax.jit
def kernel(x, data_key, ln_g, ln_b, W1, b1, W2, b2, gate_W, gate_b):
    B, C, T = x.shape
    n_groups = B // _NV

    in_specs = [
        pl.BlockSpec((_NV, C, T), lambda i: (i, 0, 0)),
        pl.BlockSpec((C, 1), lambda i: (0, 0)),
        pl.BlockSpec((C, 1), lambda i: (0, 0)),
        pl.BlockSpec((C, C), lambda i: (0, 0)),
        pl.BlockSpec((C, 1), lambda i: (0, 0)),
        pl.BlockSpec((C, C), lambda i: (0, 0)),
        pl.BlockSpec((C, 1), lambda i: (0, 0)),
        pl.BlockSpec((C, C), lambda i: (0, 0)),
        pl.BlockSpec((C, 1), lambda i: (0, 0)),
    ]

    return pl.pallas_call(
        _fused_body,
        grid=(n_groups,),
        in_specs=in_specs,
        out_specs=pl.BlockSpec((_NV, C, T), lambda i: (i, 0, 0)),
        out_shape=jax.ShapeDtypeStruct((B, C, T), x.dtype),
        compiler_params=pltpu.CompilerParams(
            dimension_semantics=("arbitrary",),
        ),
    )(x, ln_g.reshape(C, 1), ln_b.reshape(C, 1), W1, b1.reshape(C, 1),
      W2, b2.reshape(C, 1), gate_W, gate_b.reshape(C, 1))
